# baseline (speedup 1.0000x reference)
import functools

import jax
import jax.numpy as jnp
from jax import lax
from jax.experimental import pallas as pl
from jax.experimental.pallas import tpu as pltpu
from jax.experimental.pallas import tpu_sc as plsc

_BATCH = 16384
_FIELDS = 26
_DIM = 64
_B = _BATCH * _FIELDS

_NC = 2
_NS = 16
_NW = _NC * _NS
_BPW = _B // _NW
_CHUNK = 128
_NCHUNK = _BPW // _CHUNK
_K = 8

_mesh = plsc.VectorSubcoreMesh(core_axis_name="c", subcore_axis_name="s")


@functools.partial(
    pl.kernel,
    out_type=jax.ShapeDtypeStruct((_B, _DIM), jnp.float32),
    mesh=_mesh,
    scratch_types=[
        pltpu.VMEM((_NCHUNK, _CHUNK), jnp.int32),
        [pltpu.VMEM((_CHUNK, _DIM), jnp.float32) for _ in range(_K)],
        [pltpu.SemaphoreType.DMA for _ in range(_K)],
    ],
    compiler_params=pltpu.CompilerParams(use_tc_tiling_on_sc=False),
)
def _sc_gather(table_hbm, idx_hbm, out_hbm, idx_v, bufs, sems):
    wid = lax.axis_index("s") * _NC + lax.axis_index("c")
    base = wid * _BPW
    pltpu.sync_copy(idx_hbm.at[wid], idx_v)
    for b in range(_K):
        pltpu.async_copy(table_hbm.at[idx_v.at[b]], bufs[b], sems[b])

    @pl.loop(0, _NCHUNK, step=_K)
    def _group(g):
        for b in range(_K):
            i = g + b
            pltpu.make_async_copy(table_hbm.at[idx_v.at[i]], bufs[b], sems[b]).wait()
            pltpu.sync_copy(bufs[b], out_hbm.at[pl.ds(base + i * _CHUNK, _CHUNK)])

            @pl.when(i + _K < _NCHUNK)
            def _refill():
                pltpu.async_copy(table_hbm.at[idx_v.at[i + _K]], bufs[b], sems[b])


def kernel(x, weight):
    idx = x.reshape(_NW, _NCHUNK, _CHUNK).astype(jnp.int32)
    out = _sc_gather(weight, idx)
    return out.reshape(_BATCH, _FIELDS, _DIM)
